# Initial kernel scaffold; baseline (speedup 1.0000x reference)
import functools

import jax
import jax.numpy as jnp
from jax.experimental import pallas as pl
from jax.experimental.pallas import tpu as pltpu

N = 50000
E = 800000
H1 = 64
H2 = 16
HEADS = 8
EPS_BN = 1e-5

_NB = 2500
_EB = 5000

_f32 = jnp.float32


def _row_spec(b, w):
    return pl.BlockSpec((b, w), lambda i: (i, 0))


def _rep_spec(r, c):
    return pl.BlockSpec((r, c), lambda i: (0, 0))


def _conv1_body(x_ref, agg_ref, w1_ref, b1_ref, w2_ref, b2_ref,
                out_ref, s1_ref, s2_ref):
    i = pl.program_id(0)
    t = x_ref[...] + agg_ref[...]
    u = jnp.maximum(t * w1_ref[...] + b1_ref[...], 0.0)
    v = jnp.maximum(jnp.dot(u, w2_ref[...], preferred_element_type=_f32)
                    + b2_ref[...], 0.0)
    out_ref[...] = v

    @pl.when(i == 0)
    def _():
        s1_ref[...] = jnp.zeros_like(s1_ref)
        s2_ref[...] = jnp.zeros_like(s2_ref)

    s1_ref[...] += jnp.sum(v, axis=0, keepdims=True)
    s2_ref[...] += jnp.sum(v * v, axis=0, keepdims=True)


def _conv1(x2, agg1, w1, b1, w2, b2):
    return pl.pallas_call(
        _conv1_body,
        grid=(N // _NB,),
        in_specs=[_row_spec(_NB, 1), _row_spec(_NB, 1), _rep_spec(1, H1),
                  _rep_spec(1, H1), _rep_spec(H1, H1), _rep_spec(1, H1)],
        out_specs=[_row_spec(_NB, H1), _rep_spec(1, H1), _rep_spec(1, H1)],
        out_shape=[jax.ShapeDtypeStruct((N, H1), _f32),
                   jax.ShapeDtypeStruct((1, H1), _f32),
                   jax.ShapeDtypeStruct((1, H1), _f32)],
    )(x2, agg1, w1, b1, w2, b2)


def _affine_mask_body(v_ref, m_ref, a_ref, c_ref, out_ref):
    mask = (m_ref[...] > 0.0).astype(_f32)
    out_ref[...] = (a_ref[...] * v_ref[...] + c_ref[...]) * mask


def _affine_mask(v, agg0, a, c):
    return pl.pallas_call(
        _affine_mask_body,
        grid=(N // _NB,),
        in_specs=[_row_spec(_NB, H1), _row_spec(_NB, 1),
                  _rep_spec(1, H1), _rep_spec(1, H1)],
        out_specs=_row_spec(_NB, H1),
        out_shape=jax.ShapeDtypeStruct((N, H1), _f32),
    )(v, agg0, a, c)


def _affine_body(v_ref, a_ref, c_ref, out_ref):
    out_ref[...] = a_ref[...] * v_ref[...] + c_ref[...]


def _affine(v, a, c):
    return pl.pallas_call(
        _affine_body,
        grid=(N // _NB,),
        in_specs=[_row_spec(_NB, H1), _rep_spec(1, H1), _rep_spec(1, H1)],
        out_specs=_row_spec(_NB, H1),
        out_shape=jax.ShapeDtypeStruct((N, H1), _f32),
    )(v, a, c)


def _gin_mlp_body(h_ref, agg_ref, eps_ref, w1_ref, b1_ref, w2_ref, b2_ref,
                  out_ref, s1_ref, s2_ref):
    i = pl.program_id(0)
    o = (1.0 + eps_ref[0, 0]) * h_ref[...] + agg_ref[...]
    z = jnp.maximum(jnp.dot(o, w1_ref[...], preferred_element_type=_f32)
                    + b1_ref[...], 0.0)
    z2 = jnp.maximum(jnp.dot(z, w2_ref[...], preferred_element_type=_f32)
                     + b2_ref[...], 0.0)
    out_ref[...] = z2

    @pl.when(i == 0)
    def _():
        s1_ref[...] = jnp.zeros_like(s1_ref)
        s2_ref[...] = jnp.zeros_like(s2_ref)

    s1_ref[...] += jnp.sum(z2, axis=0, keepdims=True)
    s2_ref[...] += jnp.sum(z2 * z2, axis=0, keepdims=True)


def _gin_mlp(h, agg, eps, w1, b1, w2, b2):
    return pl.pallas_call(
        _gin_mlp_body,
        grid=(N // _NB,),
        in_specs=[_row_spec(_NB, H1), _row_spec(_NB, H1), _rep_spec(1, 1),
                  _rep_spec(H1, H1), _rep_spec(1, H1), _rep_spec(H1, H1),
                  _rep_spec(1, H1)],
        out_specs=[_row_spec(_NB, H1), _rep_spec(1, H1), _rep_spec(1, H1)],
        out_shape=[jax.ShapeDtypeStruct((N, H1), _f32),
                   jax.ShapeDtypeStruct((1, H1), _f32),
                   jax.ShapeDtypeStruct((1, H1), _f32)],
    )(h, agg, eps, w1, b1, w2, b2)


def _resid_body(h_ref, z_ref, a_ref, c_ref, out_ref, s1_ref, s2_ref):
    i = pl.program_id(0)
    hp = h_ref[...] + (a_ref[...] * z_ref[...] + c_ref[...])
    out_ref[...] = hp

    @pl.when(i == 0)
    def _():
        s1_ref[...] = jnp.zeros_like(s1_ref)
        s2_ref[...] = jnp.zeros_like(s2_ref)

    s1_ref[...] += jnp.sum(hp, axis=0, keepdims=True)
    s2_ref[...] += jnp.sum(hp * hp, axis=0, keepdims=True)


def _resid(h, z2, a, c):
    return pl.pallas_call(
        _resid_body,
        grid=(N // _NB,),
        in_specs=[_row_spec(_NB, H1), _row_spec(_NB, H1),
                  _rep_spec(1, H1), _rep_spec(1, H1)],
        out_specs=[_row_spec(_NB, H1), _rep_spec(1, H1), _rep_spec(1, H1)],
        out_shape=[jax.ShapeDtypeStruct((N, H1), _f32),
                   jax.ShapeDtypeStruct((1, H1), _f32),
                   jax.ShapeDtypeStruct((1, H1), _f32)],
    )(h, z2, a, c)


def _gat_proj_body(h_ref, gw_ref, watt_ref, hp_ref, al_ref, mx_ref):
    i = pl.program_id(0)
    hp = jnp.dot(h_ref[...], gw_ref[...], preferred_element_type=_f32)
    hp_ref[...] = hp
    alr = jnp.dot(hp, watt_ref[...], preferred_element_type=_f32)
    al_ref[...] = alr
    pmx = jnp.max(alr, axis=0, keepdims=True)

    @pl.when(i == 0)
    def _():
        mx_ref[...] = jnp.full_like(mx_ref, -1e30)

    mx_ref[...] = jnp.maximum(mx_ref[...], pmx)


def _gat_proj(h, gw, watt):
    return pl.pallas_call(
        _gat_proj_body,
        grid=(N // _NB,),
        in_specs=[_row_spec(_NB, H1), _rep_spec(H1, HEADS * H2),
                  _rep_spec(HEADS * H2, 2 * HEADS)],
        out_specs=[_row_spec(_NB, HEADS * H2), _row_spec(_NB, 2 * HEADS),
                   _rep_spec(1, 2 * HEADS)],
        out_shape=[jax.ShapeDtypeStruct((N, HEADS * H2), _f32),
                   jax.ShapeDtypeStruct((N, 2 * HEADS), _f32),
                   jax.ShapeDtypeStruct((1, 2 * HEADS), _f32)],
    )(h, gw, watt)


def _edge_ex_body(ad_ref, as_ref, src_ref, dst_ref, mx_ref, ex_ref):
    i = pl.program_id(0)
    alpha = ad_ref[:, :HEADS] + as_ref[:, HEADS:]
    alpha = jnp.where(alpha >= 0.0, alpha, 0.2 * alpha)
    bound = mx_ref[:, :HEADS] + mx_ref[:, HEADS:]
    pos = i * _EB + jax.lax.broadcasted_iota(jnp.int32, (_EB, 1), 0)
    valid = (src_ref[...] != dst_ref[...]) | ((pos >= E) & (pos < E + N))
    ex_ref[...] = jnp.where(valid, jnp.exp(alpha - bound), 0.0)


def _edge_ex(ad, asr, src2, dst2, mx):
    e2 = ad.shape[0]
    return pl.pallas_call(
        _edge_ex_body,
        grid=(e2 // _EB,),
        in_specs=[_row_spec(_EB, 2 * HEADS), _row_spec(_EB, 2 * HEADS),
                  _row_spec(_EB, 1), _row_spec(_EB, 1),
                  _rep_spec(1, 2 * HEADS)],
        out_specs=_row_spec(_EB, HEADS),
        out_shape=jax.ShapeDtypeStruct((e2, HEADS), _f32),
    )(ad, asr, src2, dst2, mx)


def _recip_body(d_ref, out_ref):
    out_ref[...] = 1.0 / (d_ref[...] + 1e-16)


def _recip(den):
    return pl.pallas_call(
        _recip_body,
        grid=(N // _NB,),
        in_specs=[_row_spec(_NB, HEADS)],
        out_specs=_row_spec(_NB, HEADS),
        out_shape=jax.ShapeDtypeStruct((N, HEADS), _f32),
    )(den)


def _lin1_body(g_ref, gb_ref, w_ref, b_ref, out_ref, s1_ref, s2_ref):
    i = pl.program_id(0)
    y = jnp.maximum(
        jnp.dot(g_ref[...] + gb_ref[...], w_ref[...],
                preferred_element_type=_f32) + b_ref[...], 0.0)
    out_ref[...] = y

    @pl.when(i == 0)
    def _():
        s1_ref[...] = jnp.zeros_like(s1_ref)
        s2_ref[...] = jnp.zeros_like(s2_ref)

    s1_ref[...] += jnp.sum(y, axis=0, keepdims=True)
    s2_ref[...] += jnp.sum(y * y, axis=0, keepdims=True)


def _lin1(g, gb, w, b):
    return pl.pallas_call(
        _lin1_body,
        grid=(N // _NB,),
        in_specs=[_row_spec(_NB, HEADS * H2), _rep_spec(1, HEADS * H2),
                  _rep_spec(HEADS * H2, H1), _rep_spec(1, H1)],
        out_specs=[_row_spec(_NB, H1), _rep_spec(1, H1), _rep_spec(1, H1)],
        out_shape=[jax.ShapeDtypeStruct((N, H1), _f32),
                   jax.ShapeDtypeStruct((1, H1), _f32),
                   jax.ShapeDtypeStruct((1, H1), _f32)],
    )(g, gb, w, b)


def _final_body(y_ref, a_ref, c_ref, w_ref, b_ref, out_ref):
    h = a_ref[...] * y_ref[...] + c_ref[...]
    out_ref[...] = jax.nn.sigmoid(
        jnp.dot(h, w_ref[...], preferred_element_type=_f32) + b_ref[...])


def _final(y, a, c, w, b):
    return pl.pallas_call(
        _final_body,
        grid=(N // _NB,),
        in_specs=[_row_spec(_NB, H1), _rep_spec(1, H1), _rep_spec(1, H1),
                  _rep_spec(H1, 1), _rep_spec(1, 1)],
        out_specs=_row_spec(_NB, 1),
        out_shape=jax.ShapeDtypeStruct((N, 1), _f32),
    )(y, a, c, w, b)


def _bn_affine(s1, s2, g, b):
    m = s1[0] / N
    v = s2[0] / N - m * m
    inv = g / jnp.sqrt(v + EPS_BN)
    return (inv[None, :], (b - m * inv)[None, :])


def kernel(x, edge_index, batch, c1_w1, c1_b1, c1_w2, c1_b2, c1_g, c1_bb,
           gins_w1, gins_b1, gins_w2, gins_b2, gins_g, gins_bb, gins_eps,
           bns_g, bns_b, gat_w, gat_att, gat_bias, lin1_w, lin1_b,
           bn2_g, bn2_b, lin2_w, lin2_b):
    src, dst = edge_index[0], edge_index[1]

    agg0 = jnp.zeros((N,), _f32).at[src].add(x[dst])
    agg1 = jnp.zeros((N,), _f32).at[dst].add(x[src])

    v, s1, s2 = _conv1(x[:, None], agg1[:, None], c1_w1, c1_b1[None, :],
                       c1_w2, c1_b2[None, :])
    a, c = _bn_affine(s1, s2, c1_g, c1_bb)
    h = _affine_mask(v, agg0[:, None], a, c)

    for i in range(2):
        agg = jnp.zeros((N, H1), _f32).at[dst].add(h[src])
        z2, s1, s2 = _gin_mlp(h, agg, gins_eps[i].reshape(1, 1),
                              gins_w1[i], gins_b1[i][None, :],
                              gins_w2[i], gins_b2[i][None, :])
        a, c = _bn_affine(s1, s2, gins_g[i], gins_bb[i])
        hp, s1, s2 = _resid(h, z2, a, c)
        a, c = _bn_affine(s1, s2, bns_g[i], bns_b[i])
        h = _affine(hp, a, c)

    watt = jnp.zeros((HEADS * H2, 2 * HEADS), _f32)
    att_l = gat_att[0, :, :H2]
    att_r = gat_att[0, :, H2:]
    for hh in range(HEADS):
        watt = watt.at[hh * H2:(hh + 1) * H2, hh].set(att_l[hh])
        watt = watt.at[hh * H2:(hh + 1) * H2, HEADS + hh].set(att_r[hh])

    hp, alr, mx = _gat_proj(h, gat_w, watt)

    loop = jnp.arange(N, dtype=src.dtype)
    src2 = jnp.concatenate([src, loop])
    dst2 = jnp.concatenate([dst, loop])
    e2 = E + N
    e2p = ((e2 + _EB - 1) // _EB) * _EB
    pad = e2p - e2
    src2 = jnp.concatenate([src2, jnp.zeros((pad,), src.dtype)])
    dst2 = jnp.concatenate([dst2, jnp.zeros((pad,), src.dtype)])

    ad = alr[dst2]
    asr = alr[src2]
    ex = _edge_ex(ad, asr, src2[:, None], dst2[:, None], mx)
    den = jnp.zeros((N, HEADS), _f32).at[dst2].add(ex)
    rden = _recip(den)
    aatt = ex * rden[dst2]
    msg = hp[src2].reshape(e2p, HEADS, H2) * aatt[..., None]
    gout = jnp.zeros((N, HEADS, H2), _f32).at[dst2].add(msg)
    gout = gout.reshape(N, HEADS * H2)

    y, s1, s2 = _lin1(gout, gat_bias[None, :], lin1_w, lin1_b[None, :])
    a, c = _bn_affine(s1, s2, bn2_g, bn2_b)
    probs = _final(y, a, c, lin2_w, lin2_b.reshape(1, 1))
    return probs


# speedup vs baseline: 1.3123x; 1.3123x over previous
import functools

import jax
import jax.numpy as jnp
from jax.experimental import pallas as pl
from jax.experimental.pallas import tpu as pltpu

N = 50000
E = 800000
H1 = 64
H2 = 16
HEADS = 8
EPS_BN = 1e-5

_NB = 2000
_EB = 5000

_f32 = jnp.float32


def _row_spec(b, w):
    return pl.BlockSpec((b, w), lambda i: (i, 0))


def _rep_spec(r, c):
    return pl.BlockSpec((r, c), lambda i: (0, 0))


def _conv1_body(x_ref, agg_ref, w1_ref, b1_ref, w2_ref, b2_ref,
                out_ref, s1_ref, s2_ref):
    i = pl.program_id(0)
    t = x_ref[...] + agg_ref[...]
    u = jnp.maximum(t * w1_ref[...] + b1_ref[...], 0.0)
    v = jnp.maximum(jnp.dot(u, w2_ref[...], preferred_element_type=_f32)
                    + b2_ref[...], 0.0)
    out_ref[...] = v

    @pl.when(i == 0)
    def _():
        s1_ref[...] = jnp.zeros_like(s1_ref)
        s2_ref[...] = jnp.zeros_like(s2_ref)

    s1_ref[...] += jnp.sum(v, axis=0, keepdims=True)
    s2_ref[...] += jnp.sum(v * v, axis=0, keepdims=True)


def _conv1(x2, agg1, w1, b1, w2, b2):
    return pl.pallas_call(
        _conv1_body,
        grid=(N // _NB,),
        in_specs=[_row_spec(_NB, 1), _row_spec(_NB, 1), _rep_spec(1, H1),
                  _rep_spec(1, H1), _rep_spec(H1, H1), _rep_spec(1, H1)],
        out_specs=[_row_spec(_NB, H1), _rep_spec(1, H1), _rep_spec(1, H1)],
        out_shape=[jax.ShapeDtypeStruct((N, H1), _f32),
                   jax.ShapeDtypeStruct((1, H1), _f32),
                   jax.ShapeDtypeStruct((1, H1), _f32)],
    )(x2, agg1, w1, b1, w2, b2)


def _affine_mask_body(v_ref, m_ref, a_ref, c_ref, out_ref):
    mask = (m_ref[...] > 0.0).astype(_f32)
    out_ref[...] = (a_ref[...] * v_ref[...] + c_ref[...]) * mask


def _affine_mask(v, agg0, a, c):
    return pl.pallas_call(
        _affine_mask_body,
        grid=(N // _NB,),
        in_specs=[_row_spec(_NB, H1), _row_spec(_NB, 1),
                  _rep_spec(1, H1), _rep_spec(1, H1)],
        out_specs=_row_spec(_NB, H1),
        out_shape=jax.ShapeDtypeStruct((N, H1), _f32),
    )(v, agg0, a, c)


def _affine_body(v_ref, a_ref, c_ref, out_ref):
    out_ref[...] = a_ref[...] * v_ref[...] + c_ref[...]


def _affine(v, a, c):
    return pl.pallas_call(
        _affine_body,
        grid=(N // _NB,),
        in_specs=[_row_spec(_NB, H1), _rep_spec(1, H1), _rep_spec(1, H1)],
        out_specs=_row_spec(_NB, H1),
        out_shape=jax.ShapeDtypeStruct((N, H1), _f32),
    )(v, a, c)


def _gin_mlp_body(h_ref, agg_ref, eps_ref, w1_ref, b1_ref, w2_ref, b2_ref,
                  out_ref, s1_ref, s2_ref):
    i = pl.program_id(0)
    o = (1.0 + eps_ref[0, 0]) * h_ref[...] + agg_ref[...]
    z = jnp.maximum(jnp.dot(o, w1_ref[...], preferred_element_type=_f32)
                    + b1_ref[...], 0.0)
    z2 = jnp.maximum(jnp.dot(z, w2_ref[...], preferred_element_type=_f32)
                     + b2_ref[...], 0.0)
    out_ref[...] = z2

    @pl.when(i == 0)
    def _():
        s1_ref[...] = jnp.zeros_like(s1_ref)
        s2_ref[...] = jnp.zeros_like(s2_ref)

    s1_ref[...] += jnp.sum(z2, axis=0, keepdims=True)
    s2_ref[...] += jnp.sum(z2 * z2, axis=0, keepdims=True)


def _gin_mlp(h, agg, eps, w1, b1, w2, b2):
    return pl.pallas_call(
        _gin_mlp_body,
        grid=(N // _NB,),
        in_specs=[_row_spec(_NB, H1), _row_spec(_NB, H1), _rep_spec(1, 1),
                  _rep_spec(H1, H1), _rep_spec(1, H1), _rep_spec(H1, H1),
                  _rep_spec(1, H1)],
        out_specs=[_row_spec(_NB, H1), _rep_spec(1, H1), _rep_spec(1, H1)],
        out_shape=[jax.ShapeDtypeStruct((N, H1), _f32),
                   jax.ShapeDtypeStruct((1, H1), _f32),
                   jax.ShapeDtypeStruct((1, H1), _f32)],
    )(h, agg, eps, w1, b1, w2, b2)


def _resid_body(h_ref, z_ref, a_ref, c_ref, out_ref, s1_ref, s2_ref):
    i = pl.program_id(0)
    hp = h_ref[...] + (a_ref[...] * z_ref[...] + c_ref[...])
    out_ref[...] = hp

    @pl.when(i == 0)
    def _():
        s1_ref[...] = jnp.zeros_like(s1_ref)
        s2_ref[...] = jnp.zeros_like(s2_ref)

    s1_ref[...] += jnp.sum(hp, axis=0, keepdims=True)
    s2_ref[...] += jnp.sum(hp * hp, axis=0, keepdims=True)


def _resid(h, z2, a, c):
    return pl.pallas_call(
        _resid_body,
        grid=(N // _NB,),
        in_specs=[_row_spec(_NB, H1), _row_spec(_NB, H1),
                  _rep_spec(1, H1), _rep_spec(1, H1)],
        out_specs=[_row_spec(_NB, H1), _rep_spec(1, H1), _rep_spec(1, H1)],
        out_shape=[jax.ShapeDtypeStruct((N, H1), _f32),
                   jax.ShapeDtypeStruct((1, H1), _f32),
                   jax.ShapeDtypeStruct((1, H1), _f32)],
    )(h, z2, a, c)


def _gat_proj_body(h_ref, gw_ref, watt_ref, hp_ref, al_ref, mx_ref):
    i = pl.program_id(0)
    hp = jnp.dot(h_ref[...], gw_ref[...], preferred_element_type=_f32)
    hp_ref[...] = hp
    alr = jnp.dot(hp, watt_ref[...], preferred_element_type=_f32)
    al_ref[...] = alr
    pmx = jnp.max(alr, axis=0, keepdims=True)

    @pl.when(i == 0)
    def _():
        mx_ref[...] = jnp.full_like(mx_ref, -1e30)

    mx_ref[...] = jnp.maximum(mx_ref[...], pmx)


def _gat_proj(h, gw, watt):
    return pl.pallas_call(
        _gat_proj_body,
        grid=(N // _NB,),
        in_specs=[_row_spec(_NB, H1), _rep_spec(H1, HEADS * H2),
                  _rep_spec(HEADS * H2, 2 * HEADS)],
        out_specs=[_row_spec(_NB, HEADS * H2), _row_spec(_NB, 2 * HEADS),
                   _rep_spec(1, 2 * HEADS)],
        out_shape=[jax.ShapeDtypeStruct((N, HEADS * H2), _f32),
                   jax.ShapeDtypeStruct((N, 2 * HEADS), _f32),
                   jax.ShapeDtypeStruct((1, 2 * HEADS), _f32)],
    )(h, gw, watt)


def _edge_ex_body(ad_ref, as_ref, src_ref, dst_ref, mx_ref, ex_ref):
    i = pl.program_id(0)
    alpha = ad_ref[:, :HEADS] + as_ref[:, HEADS:]
    alpha = jnp.where(alpha >= 0.0, alpha, 0.2 * alpha)
    bound = mx_ref[:, :HEADS] + mx_ref[:, HEADS:]
    pos = i * _EB + jax.lax.broadcasted_iota(jnp.int32, (_EB, 1), 0)
    valid = (src_ref[...] != dst_ref[...]) | ((pos >= E) & (pos < E + N))
    ex_ref[...] = jnp.where(valid, jnp.exp(alpha - bound), 0.0)


def _edge_ex(ad, asr, src2, dst2, mx):
    e2 = ad.shape[0]
    return pl.pallas_call(
        _edge_ex_body,
        grid=(e2 // _EB,),
        in_specs=[_row_spec(_EB, 2 * HEADS), _row_spec(_EB, 2 * HEADS),
                  _row_spec(_EB, 1), _row_spec(_EB, 1),
                  _rep_spec(1, 2 * HEADS)],
        out_specs=_row_spec(_EB, HEADS),
        out_shape=jax.ShapeDtypeStruct((e2, HEADS), _f32),
    )(ad, asr, src2, dst2, mx)


def _recip_body(d_ref, out_ref):
    out_ref[...] = 1.0 / (d_ref[...] + 1e-16)


def _recip(den):
    return pl.pallas_call(
        _recip_body,
        grid=(N // _NB,),
        in_specs=[_row_spec(_NB, HEADS)],
        out_specs=_row_spec(_NB, HEADS),
        out_shape=jax.ShapeDtypeStruct((N, HEADS), _f32),
    )(den)


def _lin1_body(g_ref, gb_ref, w_ref, b_ref, out_ref, s1_ref, s2_ref):
    i = pl.program_id(0)
    y = jnp.maximum(
        jnp.dot(g_ref[...] + gb_ref[...], w_ref[...],
                preferred_element_type=_f32) + b_ref[...], 0.0)
    out_ref[...] = y

    @pl.when(i == 0)
    def _():
        s1_ref[...] = jnp.zeros_like(s1_ref)
        s2_ref[...] = jnp.zeros_like(s2_ref)

    s1_ref[...] += jnp.sum(y, axis=0, keepdims=True)
    s2_ref[...] += jnp.sum(y * y, axis=0, keepdims=True)


def _lin1(g, gb, w, b):
    return pl.pallas_call(
        _lin1_body,
        grid=(N // _NB,),
        in_specs=[_row_spec(_NB, HEADS * H2), _rep_spec(1, HEADS * H2),
                  _rep_spec(HEADS * H2, H1), _rep_spec(1, H1)],
        out_specs=[_row_spec(_NB, H1), _rep_spec(1, H1), _rep_spec(1, H1)],
        out_shape=[jax.ShapeDtypeStruct((N, H1), _f32),
                   jax.ShapeDtypeStruct((1, H1), _f32),
                   jax.ShapeDtypeStruct((1, H1), _f32)],
    )(g, gb, w, b)


def _final_body(y_ref, a_ref, c_ref, w_ref, b_ref, out_ref):
    h = a_ref[...] * y_ref[...] + c_ref[...]
    out_ref[...] = jax.nn.sigmoid(
        jnp.dot(h, w_ref[...], preferred_element_type=_f32) + b_ref[...])


def _final(y, a, c, w, b):
    return pl.pallas_call(
        _final_body,
        grid=(N // _NB,),
        in_specs=[_row_spec(_NB, H1), _rep_spec(1, H1), _rep_spec(1, H1),
                  _rep_spec(H1, 1), _rep_spec(1, 1)],
        out_specs=_row_spec(_NB, 1),
        out_shape=jax.ShapeDtypeStruct((N, 1), _f32),
    )(y, a, c, w, b)


def _bn_affine(s1, s2, g, b):
    m = s1[0] / N
    v = s2[0] / N - m * m
    inv = g / jnp.sqrt(v + EPS_BN)
    return (inv[None, :], (b - m * inv)[None, :])


def kernel(x, edge_index, batch, c1_w1, c1_b1, c1_w2, c1_b2, c1_g, c1_bb,
           gins_w1, gins_b1, gins_w2, gins_b2, gins_g, gins_bb, gins_eps,
           bns_g, bns_b, gat_w, gat_att, gat_bias, lin1_w, lin1_b,
           bn2_g, bn2_b, lin2_w, lin2_b):
    src, dst = edge_index[0], edge_index[1]

    agg0 = jnp.zeros((N,), _f32).at[src].add(x[dst])
    agg1 = jnp.zeros((N,), _f32).at[dst].add(x[src])

    v, s1, s2 = _conv1(x[:, None], agg1[:, None], c1_w1, c1_b1[None, :],
                       c1_w2, c1_b2[None, :])
    a, c = _bn_affine(s1, s2, c1_g, c1_bb)
    h = _affine_mask(v, agg0[:, None], a, c)

    for i in range(2):
        agg = jnp.zeros((N, H1), _f32).at[dst].add(h[src])
        z2, s1, s2 = _gin_mlp(h, agg, gins_eps[i].reshape(1, 1),
                              gins_w1[i], gins_b1[i][None, :],
                              gins_w2[i], gins_b2[i][None, :])
        a, c = _bn_affine(s1, s2, gins_g[i], gins_bb[i])
        hp, s1, s2 = _resid(h, z2, a, c)
        a, c = _bn_affine(s1, s2, bns_g[i], bns_b[i])
        h = _affine(hp, a, c)

    watt = jnp.zeros((HEADS * H2, 2 * HEADS), _f32)
    att_l = gat_att[0, :, :H2]
    att_r = gat_att[0, :, H2:]
    for hh in range(HEADS):
        watt = watt.at[hh * H2:(hh + 1) * H2, hh].set(att_l[hh])
        watt = watt.at[hh * H2:(hh + 1) * H2, HEADS + hh].set(att_r[hh])

    hp, alr, mx = _gat_proj(h, gat_w, watt)

    loop = jnp.arange(N, dtype=src.dtype)
    src2 = jnp.concatenate([src, loop])
    dst2 = jnp.concatenate([dst, loop])
    e2 = E + N
    e2p = ((e2 + _EB - 1) // _EB) * _EB
    pad = e2p - e2
    src2 = jnp.concatenate([src2, jnp.zeros((pad,), src.dtype)])
    dst2 = jnp.concatenate([dst2, jnp.zeros((pad,), src.dtype)])

    ad = alr[dst2]
    asr = alr[src2]
    ex = _edge_ex(ad, asr, src2[:, None], dst2[:, None], mx)
    den = jnp.zeros((N, HEADS), _f32).at[dst2].add(ex)
    rden = _recip(den)
    aatt = ex * rden[dst2]
    msg = hp[src2].reshape(e2p, HEADS, H2) * aatt[..., None]
    gout = jnp.zeros((N, HEADS, H2), _f32).at[dst2].add(msg)
    gout = gout.reshape(N, HEADS * H2)

    y, s1, s2 = _lin1(gout, gat_bias[None, :], lin1_w, lin1_b[None, :])
    a, c = _bn_affine(s1, s2, bn2_g, bn2_b)
    probs = _final(y, a, c, lin2_w, lin2_b.reshape(1, 1))
    return probs
